# Initial kernel scaffold; baseline (speedup 1.0000x reference)
import jax
import jax.numpy as jnp
from jax.experimental import pallas as pl


def kernel(x, lower_neighborhood_indices, lower_neighborhood_values, upper_neighborhood_indices, upper_neighborhood_values, W_irr, att_irr, W_sol, att_sol, W_har):
    raise NotImplementedError("write your pallas kernel here")


# speedup vs baseline: 6.6106x; 6.6106x over previous
import functools

import jax
import jax.numpy as jnp
from jax import lax
from jax.experimental import pallas as pl
from jax.experimental.pallas import tpu as pltpu
from jax.experimental.pallas import tpu_sc as plsc

N = 10000
D = 128
LANES = 16
CHUNK = 128
NUM_TILES = 16
ROWBLK = 80
NUM_ROWBLKS = N // ROWBLK


def _stage1(x, W_irr, att_irr, W_sol, att_sol):
    BLK = 1000
    grid = N // BLK

    def body(x_ref, wi_ref, ai_ref, ws_ref, as_ref,
             xmi_ref, xms_ref, ais_ref, ait_ref, ass_ref, ast_ref):
        xb = x_ref[...]
        xmi = jnp.dot(xb, wi_ref[...], preferred_element_type=jnp.float32)
        xms = jnp.dot(xb, ws_ref[...], preferred_element_type=jnp.float32)
        xmi_ref[...] = xmi
        xms_ref[...] = xms
        ais_ref[...] = jnp.dot(xmi, ai_ref[:D, :], preferred_element_type=jnp.float32)
        ait_ref[...] = jnp.dot(xmi, ai_ref[D:, :], preferred_element_type=jnp.float32)
        ass_ref[...] = jnp.dot(xms, as_ref[:D, :], preferred_element_type=jnp.float32)
        ast_ref[...] = jnp.dot(xms, as_ref[D:, :], preferred_element_type=jnp.float32)

    full = pl.BlockSpec((D, D), lambda i: (0, 0))
    att = pl.BlockSpec((2 * D, 1), lambda i: (0, 0))
    rows = pl.BlockSpec((BLK, D), lambda i: (i, 0))
    col = pl.BlockSpec((BLK, 1), lambda i: (i, 0))
    return pl.pallas_call(
        body,
        grid=(grid,),
        in_specs=[rows, full, att, full, att],
        out_specs=[rows, rows, col, col, col, col],
        out_shape=[
            jax.ShapeDtypeStruct((N, D), jnp.float32),
            jax.ShapeDtypeStruct((N, D), jnp.float32),
            jax.ShapeDtypeStruct((N, 1), jnp.float32),
            jax.ShapeDtypeStruct((N, 1), jnp.float32),
            jax.ShapeDtypeStruct((N, 1), jnp.float32),
            jax.ShapeDtypeStruct((N, 1), jnp.float32),
        ],
    )(x, W_irr, att_irr, W_sol, att_sol)


def _stage3(x, w_har_t, s_irr, s_sol):
    BLK = 1000
    grid = N // BLK

    def body(x_ref, wt_ref, si_ref, ss_ref, out_ref):
        h = jnp.dot(x_ref[...], wt_ref[...], preferred_element_type=jnp.float32)
        out_ref[...] = jnp.maximum(
            si_ref[...] + ss_ref[...] + h * (1.0 + 1e-06), 0.0)

    full = pl.BlockSpec((D, D), lambda i: (0, 0))
    rows = pl.BlockSpec((BLK, D), lambda i: (i, 0))
    return pl.pallas_call(
        body,
        grid=(grid,),
        in_specs=[rows, full, rows, rows],
        out_specs=rows,
        out_shape=jax.ShapeDtypeStruct((N, D), jnp.float32),
    )(x, w_har_t, s_irr, s_sol)


def _sc_conv_pair(xm_irr, xm_sol, ais, ait, ass, ast,
                  lt, ls, lv, ut, us, uv, chunks_per_tile):
    T = chunks_per_tile

    def process(xm_hbm, asrc_hbm, atgt_hbm, tgt_hbm, src_hbm, vals_hbm,
                out_hbm, acc, a_src_v, a_tgt_v, src_v, tgt_v, vals_v,
                w_v, rows_v, tile):
        pltpu.sync_copy(asrc_hbm, a_src_v)
        pltpu.sync_copy(atgt_hbm, a_tgt_v)

        z16 = jnp.zeros((LANES,), jnp.float32)

        def zrow(r, carry):
            for j in range(D // LANES):
                rows_v[r, pl.ds(j * LANES, LANES)] = z16
            return carry

        lax.fori_loop(0, ROWBLK, zrow, 0)
        for i in range(8):
            b = tile + i * NUM_TILES
            @pl.when(b < NUM_ROWBLKS)
            def _():
                pltpu.sync_copy(rows_v.at[pl.ds(0, ROWBLK)],
                                acc.at[pl.ds(b * ROWBLK, ROWBLK)])
        plsc.subcore_barrier()

        def chunk_body(i, carry):
            base = (tile * T + i) * CHUNK
            pltpu.sync_copy(tgt_hbm.at[pl.ds(base, CHUNK)], tgt_v)
            pltpu.sync_copy(src_hbm.at[pl.ds(base, CHUNK)], src_v)
            pltpu.sync_copy(vals_hbm.at[pl.ds(base, CHUNK)], vals_v)
            pltpu.sync_copy(xm_hbm.at[src_v], rows_v)
            for g in range(CHUNK // LANES):
                sl = pl.ds(g * LANES, LANES)
                za = (plsc.load_gather(a_src_v, [src_v[sl]])
                      + plsc.load_gather(a_tgt_v, [tgt_v[sl]]))
                att = jnp.where(za > 0.0, za,
                                jnp.exp(jnp.minimum(za, 0.0)) - 1.0)
                w_v[sl] = att * vals_v[sl]

            def scale(g, c2):
                wvec = w_v[pl.ds(g * LANES, LANES)]
                for k2 in range(LANES):
                    wb = jnp.full((LANES,), wvec[k2], jnp.float32)
                    k = g * LANES + k2
                    for j in range(D // LANES):
                        sj = pl.ds(j * LANES, LANES)
                        rows_v[k, sj] = rows_v[k, sj] * wb
                return c2

            lax.fori_loop(0, CHUNK // LANES, scale, 0)
            pltpu.sync_copy(rows_v, acc.at[tgt_v], add=True)
            return carry

        lax.fori_loop(0, T, chunk_body, 0)
        plsc.subcore_barrier()

        for i in range(8):
            b = tile + i * NUM_TILES
            @pl.when(b < NUM_ROWBLKS)
            def _():
                pltpu.sync_copy(acc.at[pl.ds(b * ROWBLK, ROWBLK)],
                                rows_v.at[pl.ds(0, ROWBLK)])
                pltpu.sync_copy(rows_v.at[pl.ds(0, ROWBLK)],
                                out_hbm.at[pl.ds(b * ROWBLK, ROWBLK)])

    def body(xm_irr_h, xm_sol_h, ais_h, ait_h, ass_h, ast_h,
             lt_h, ls_h, lv_h, ut_h, us_h, uv_h,
             out_irr, out_sol, acc, a_src_v, a_tgt_v,
             src_v, tgt_v, vals_v, w_v, rows_v):
        c = lax.axis_index("c")
        tile = lax.axis_index("s")

        @pl.when(c == 0)
        def _():
            process(xm_irr_h, ais_h, ait_h, lt_h, ls_h, lv_h, out_irr,
                    acc, a_src_v, a_tgt_v, src_v, tgt_v, vals_v, w_v,
                    rows_v, tile)

        @pl.when(c == 1)
        def _():
            process(xm_sol_h, ass_h, ast_h, ut_h, us_h, uv_h, out_sol,
                    acc, a_src_v, a_tgt_v, src_v, tgt_v, vals_v, w_v,
                    rows_v, tile)

    mesh = plsc.VectorSubcoreMesh(core_axis_name="c", subcore_axis_name="s")
    f = pl.kernel(
        body,
        out_type=[
            jax.ShapeDtypeStruct((N, D), jnp.float32),
            jax.ShapeDtypeStruct((N, D), jnp.float32),
        ],
        mesh=mesh,
        compiler_params=pltpu.CompilerParams(needs_layout_passes=False),
        scratch_types=[
            pltpu.VMEM_SHARED((N, D), jnp.float32),
            pltpu.VMEM((N,), jnp.float32),
            pltpu.VMEM((N,), jnp.float32),
            pltpu.VMEM((CHUNK,), jnp.int32),
            pltpu.VMEM((CHUNK,), jnp.int32),
            pltpu.VMEM((CHUNK,), jnp.float32),
            pltpu.VMEM((CHUNK,), jnp.float32),
            pltpu.VMEM((CHUNK, D), jnp.float32),
        ],
    )
    return f(xm_irr, xm_sol, ais, ait, ass, ast, lt, ls, lv, ut, us, uv)


def kernel(x, lower_neighborhood_indices, lower_neighborhood_values,
           upper_neighborhood_indices, upper_neighborhood_values,
           W_irr, att_irr, W_sol, att_sol, W_har):
    E = lower_neighborhood_values.shape[0]
    per_tile = -(-E // (NUM_TILES * CHUNK)) * CHUNK
    EP = per_tile * NUM_TILES
    pad = EP - E

    xm_irr, xm_sol, ais, ait, ass, ast = _stage1(
        x, W_irr, att_irr, W_sol, att_sol)

    def prep(idx, vals):
        t = jnp.pad(idx[0], (0, pad))
        s = jnp.pad(idx[1], (0, pad))
        v = jnp.pad(vals, (0, pad))
        return t, s, v

    lt, ls, lv = prep(lower_neighborhood_indices, lower_neighborhood_values)
    ut, us, uv = prep(upper_neighborhood_indices, upper_neighborhood_values)

    s_irr, s_sol = _sc_conv_pair(
        xm_irr, xm_sol,
        ais.reshape(N), ait.reshape(N), ass.reshape(N), ast.reshape(N),
        lt, ls, lv, ut, us, uv, per_tile // CHUNK)

    return _stage3(x, W_har.T, s_irr, s_sol)


# speedup vs baseline: 8.8562x; 1.3397x over previous
import functools

import jax
import jax.numpy as jnp
from jax import lax
from jax.experimental import pallas as pl
from jax.experimental.pallas import tpu as pltpu
from jax.experimental.pallas import tpu_sc as plsc

N = 10000
D = 128
LANES = 16
CHUNK = 96
NUM_TILES = 16
ROWBLK = 80
NUM_ROWBLKS = N // ROWBLK


def _stage1(x, W_irr, att_irr, W_sol, att_sol):
    BLK = 1000
    grid = N // BLK

    def body(x_ref, wi_ref, ai_ref, ws_ref, as_ref,
             xmi_ref, xms_ref, ais_ref, ait_ref, ass_ref, ast_ref):
        xb = x_ref[...]
        xmi = jnp.dot(xb, wi_ref[...], preferred_element_type=jnp.float32)
        xms = jnp.dot(xb, ws_ref[...], preferred_element_type=jnp.float32)
        xmi_ref[...] = xmi
        xms_ref[...] = xms
        ais_ref[...] = jnp.dot(xmi, ai_ref[:D, :], preferred_element_type=jnp.float32)
        ait_ref[...] = jnp.dot(xmi, ai_ref[D:, :], preferred_element_type=jnp.float32)
        ass_ref[...] = jnp.dot(xms, as_ref[:D, :], preferred_element_type=jnp.float32)
        ast_ref[...] = jnp.dot(xms, as_ref[D:, :], preferred_element_type=jnp.float32)

    full = pl.BlockSpec((D, D), lambda i: (0, 0))
    att = pl.BlockSpec((2 * D, 1), lambda i: (0, 0))
    rows = pl.BlockSpec((BLK, D), lambda i: (i, 0))
    col = pl.BlockSpec((BLK, 1), lambda i: (i, 0))
    return pl.pallas_call(
        body,
        grid=(grid,),
        in_specs=[rows, full, att, full, att],
        out_specs=[rows, rows, col, col, col, col],
        out_shape=[
            jax.ShapeDtypeStruct((N, D), jnp.float32),
            jax.ShapeDtypeStruct((N, D), jnp.float32),
            jax.ShapeDtypeStruct((N, 1), jnp.float32),
            jax.ShapeDtypeStruct((N, 1), jnp.float32),
            jax.ShapeDtypeStruct((N, 1), jnp.float32),
            jax.ShapeDtypeStruct((N, 1), jnp.float32),
        ],
    )(x, W_irr, att_irr, W_sol, att_sol)


def _stage3(x, w_har_t, s_irr, s_sol):
    BLK = 1000
    grid = N // BLK

    def body(x_ref, wt_ref, si_ref, ss_ref, out_ref):
        h = jnp.dot(x_ref[...], wt_ref[...], preferred_element_type=jnp.float32)
        out_ref[...] = jnp.maximum(
            si_ref[...] + ss_ref[...] + h * (1.0 + 1e-06), 0.0)

    full = pl.BlockSpec((D, D), lambda i: (0, 0))
    rows = pl.BlockSpec((BLK, D), lambda i: (i, 0))
    return pl.pallas_call(
        body,
        grid=(grid,),
        in_specs=[rows, full, rows, rows],
        out_specs=rows,
        out_shape=jax.ShapeDtypeStruct((N, D), jnp.float32),
    )(x, w_har_t, s_irr, s_sol)


def _sc_conv_pair(xm_irr, xm_sol, ais, ait, ass, ast,
                  lt, ls, lv, ut, us, uv, chunks_per_tile):
    T = chunks_per_tile

    def process(xm_hbm, asrc_hbm, atgt_hbm, tgt_hbm, src_hbm, vals_hbm,
                out_hbm, acc, a_src_v, a_tgt_v,
                src0, tgt0, vals0, sidx0, rows0,
                src1, tgt1, vals1, sidx1, rows1,
                w_v, sem_g0, sem_g1, sem_s0, sem_s1, tile):
        pltpu.sync_copy(asrc_hbm, a_src_v)
        pltpu.sync_copy(atgt_hbm, a_tgt_v)

        z16 = jnp.zeros((LANES,), jnp.float32)

        def zrow(r, carry):
            for j in range(D // LANES):
                rows0[r, pl.ds(j * LANES, LANES)] = z16
            return carry

        lax.fori_loop(0, ROWBLK, zrow, 0)
        for i in range(8):
            b = tile + i * NUM_TILES
            @pl.when(b < NUM_ROWBLKS)
            def _():
                pltpu.sync_copy(rows0.at[pl.ds(0, ROWBLK)],
                                acc.at[pl.ds(b * ROWBLK, ROWBLK)])
        plsc.subcore_barrier()

        def load_idx(c, s_v, t_v, v_v):
            base = (tile * T + c) * CHUNK
            pltpu.sync_copy(tgt_hbm.at[pl.ds(base, CHUNK)], t_v)
            pltpu.sync_copy(src_hbm.at[pl.ds(base, CHUNK)], s_v)
            pltpu.sync_copy(vals_hbm.at[pl.ds(base, CHUNK)], v_v)

        def compute_w(s_v, t_v, v_v, sidx):
            for g in range(CHUNK // LANES):
                sl = pl.ds(g * LANES, LANES)
                za = (plsc.load_gather(a_src_v, [s_v[sl]])
                      + plsc.load_gather(a_tgt_v, [t_v[sl]]))
                att = jnp.where(za > 0.0, za,
                                jnp.exp(jnp.minimum(za, 0.0)) - 1.0)
                w_v[sl] = att * v_v[sl]
                sidx[sl] = t_v[sl]

        def scale_rows(rows):
            def scale(g, c2):
                wvec = w_v[pl.ds(g * LANES, LANES)]
                for k2 in range(LANES):
                    wb = jnp.full((LANES,), wvec[k2], jnp.float32)
                    k = g * LANES + k2
                    for j in range(D // LANES):
                        sj = pl.ds(j * LANES, LANES)
                        rows[k, sj] = rows[k, sj] * wb
                return c2
            lax.fori_loop(0, CHUNK // LANES, scale, 0)

        load_idx(0, src0, tgt0, vals0)
        load_idx(1, src1, tgt1, vals1)
        pltpu.async_copy(xm_hbm.at[src0], rows0, sem_g0)

        def pipe_body(i, carry):
            @pl.when(i > 0)
            def _():
                pltpu.make_async_copy(rows1, acc.at[sidx1], sem_s1).wait()
            pltpu.async_copy(xm_hbm.at[src1], rows1, sem_g1)
            compute_w(src0, tgt0, vals0, sidx0)
            pltpu.make_async_copy(xm_hbm.at[src0], rows0, sem_g0).wait()
            scale_rows(rows0)
            pltpu.async_copy(rows0, acc.at[sidx0], sem_s0, add=True)

            @pl.when(i < T // 2 - 1)
            def _():
                load_idx(2 * i + 2, src0, tgt0, vals0)

            pltpu.make_async_copy(rows0, acc.at[sidx0], sem_s0).wait()

            @pl.when(i < T // 2 - 1)
            def _():
                pltpu.async_copy(xm_hbm.at[src0], rows0, sem_g0)
            compute_w(src1, tgt1, vals1, sidx1)
            pltpu.make_async_copy(xm_hbm.at[src1], rows1, sem_g1).wait()
            scale_rows(rows1)
            pltpu.async_copy(rows1, acc.at[sidx1], sem_s1, add=True)

            @pl.when(i < T // 2 - 1)
            def _():
                load_idx(2 * i + 3, src1, tgt1, vals1)
            return carry

        lax.fori_loop(0, T // 2, pipe_body, 0)
        pltpu.make_async_copy(rows1, acc.at[sidx1], sem_s1).wait()
        plsc.subcore_barrier()

        for i in range(8):
            b = tile + i * NUM_TILES
            @pl.when(b < NUM_ROWBLKS)
            def _():
                pltpu.sync_copy(acc.at[pl.ds(b * ROWBLK, ROWBLK)],
                                rows0.at[pl.ds(0, ROWBLK)])
                pltpu.sync_copy(rows0.at[pl.ds(0, ROWBLK)],
                                out_hbm.at[pl.ds(b * ROWBLK, ROWBLK)])

    def body(xm_irr_h, xm_sol_h, ais_h, ait_h, ass_h, ast_h,
             lt_h, ls_h, lv_h, ut_h, us_h, uv_h,
             out_irr, out_sol, acc, a_src_v, a_tgt_v,
             src0, tgt0, vals0, sidx0, rows0,
             src1, tgt1, vals1, sidx1, rows1,
             w_v, sem_g0, sem_g1, sem_s0, sem_s1):
        c = lax.axis_index("c")
        tile = lax.axis_index("s")

        @pl.when(c == 0)
        def _():
            process(xm_irr_h, ais_h, ait_h, lt_h, ls_h, lv_h, out_irr,
                    acc, a_src_v, a_tgt_v,
                    src0, tgt0, vals0, sidx0, rows0,
                    src1, tgt1, vals1, sidx1, rows1,
                    w_v, sem_g0, sem_g1, sem_s0, sem_s1, tile)

        @pl.when(c == 1)
        def _():
            process(xm_sol_h, ass_h, ast_h, ut_h, us_h, uv_h, out_sol,
                    acc, a_src_v, a_tgt_v,
                    src0, tgt0, vals0, sidx0, rows0,
                    src1, tgt1, vals1, sidx1, rows1,
                    w_v, sem_g0, sem_g1, sem_s0, sem_s1, tile)

    mesh = plsc.VectorSubcoreMesh(core_axis_name="c", subcore_axis_name="s")
    f = pl.kernel(
        body,
        out_type=[
            jax.ShapeDtypeStruct((N, D), jnp.float32),
            jax.ShapeDtypeStruct((N, D), jnp.float32),
        ],
        mesh=mesh,
        compiler_params=pltpu.CompilerParams(needs_layout_passes=False),
        scratch_types=[
            pltpu.VMEM_SHARED((N, D), jnp.float32),
            pltpu.VMEM((N,), jnp.float32),
            pltpu.VMEM((N,), jnp.float32),
            pltpu.VMEM((CHUNK,), jnp.int32),
            pltpu.VMEM((CHUNK,), jnp.int32),
            pltpu.VMEM((CHUNK,), jnp.float32),
            pltpu.VMEM((CHUNK,), jnp.int32),
            pltpu.VMEM((CHUNK, D), jnp.float32),
            pltpu.VMEM((CHUNK,), jnp.int32),
            pltpu.VMEM((CHUNK,), jnp.int32),
            pltpu.VMEM((CHUNK,), jnp.float32),
            pltpu.VMEM((CHUNK,), jnp.int32),
            pltpu.VMEM((CHUNK, D), jnp.float32),
            pltpu.VMEM((CHUNK,), jnp.float32),
            pltpu.SemaphoreType.DMA,
            pltpu.SemaphoreType.DMA,
            pltpu.SemaphoreType.DMA,
            pltpu.SemaphoreType.DMA,
        ],
    )
    return f(xm_irr, xm_sol, ais, ait, ass, ast, lt, ls, lv, ut, us, uv)


def kernel(x, lower_neighborhood_indices, lower_neighborhood_values,
           upper_neighborhood_indices, upper_neighborhood_values,
           W_irr, att_irr, W_sol, att_sol, W_har):
    E = lower_neighborhood_values.shape[0]
    chunks_per_tile = -(-E // (NUM_TILES * CHUNK))
    chunks_per_tile += chunks_per_tile % 2
    per_tile = chunks_per_tile * CHUNK
    EP = per_tile * NUM_TILES
    pad = EP - E

    xm_irr, xm_sol, ais, ait, ass, ast = _stage1(
        x, W_irr, att_irr, W_sol, att_sol)

    def prep(idx, vals):
        t = jnp.pad(idx[0], (0, pad))
        s = jnp.pad(idx[1], (0, pad))
        v = jnp.pad(vals, (0, pad))
        return t, s, v

    lt, ls, lv = prep(lower_neighborhood_indices, lower_neighborhood_values)
    ut, us, uv = prep(upper_neighborhood_indices, upper_neighborhood_values)

    s_irr, s_sol = _sc_conv_pair(
        xm_irr, xm_sol,
        ais.reshape(N), ait.reshape(N), ass.reshape(N), ast.reshape(N),
        lt, ls, lv, ut, us, uv, per_tile // CHUNK)

    return _stage3(x, W_har.T, s_irr, s_sol)


# speedup vs baseline: 10.4682x; 1.1820x over previous
import functools

import jax
import jax.numpy as jnp
from jax import lax
from jax.experimental import pallas as pl
from jax.experimental.pallas import tpu as pltpu
from jax.experimental.pallas import tpu_sc as plsc

N = 10000
D = 128
LANES = 16
CHUNK = 96
NUM_TILES = 16
ROWBLK = 80
NUM_ROWBLKS = N // ROWBLK


def _stage1(x, W_irr, att_irr, W_sol, att_sol):
    BLK = 1000
    grid = N // BLK

    def body(x_ref, wi_ref, ai_ref, ws_ref, as_ref,
             xmi_ref, xms_ref, ais_ref, ait_ref, ass_ref, ast_ref):
        xb = x_ref[...]
        xmi = jnp.dot(xb, wi_ref[...], preferred_element_type=jnp.float32)
        xms = jnp.dot(xb, ws_ref[...], preferred_element_type=jnp.float32)
        xmi_ref[...] = xmi
        xms_ref[...] = xms
        ais_ref[...] = jnp.dot(xmi, ai_ref[:D, :], preferred_element_type=jnp.float32)
        ait_ref[...] = jnp.dot(xmi, ai_ref[D:, :], preferred_element_type=jnp.float32)
        ass_ref[...] = jnp.dot(xms, as_ref[:D, :], preferred_element_type=jnp.float32)
        ast_ref[...] = jnp.dot(xms, as_ref[D:, :], preferred_element_type=jnp.float32)

    full = pl.BlockSpec((D, D), lambda i: (0, 0))
    att = pl.BlockSpec((2 * D, 1), lambda i: (0, 0))
    rows = pl.BlockSpec((BLK, D), lambda i: (i, 0))
    col = pl.BlockSpec((BLK, 1), lambda i: (i, 0))
    return pl.pallas_call(
        body,
        grid=(grid,),
        in_specs=[rows, full, att, full, att],
        out_specs=[rows, rows, col, col, col, col],
        out_shape=[
            jax.ShapeDtypeStruct((N, D), jnp.float32),
            jax.ShapeDtypeStruct((N, D), jnp.float32),
            jax.ShapeDtypeStruct((N, 1), jnp.float32),
            jax.ShapeDtypeStruct((N, 1), jnp.float32),
            jax.ShapeDtypeStruct((N, 1), jnp.float32),
            jax.ShapeDtypeStruct((N, 1), jnp.float32),
        ],
    )(x, W_irr, att_irr, W_sol, att_sol)


def _stage3(x, w_har_t, s_irr, s_sol):
    BLK = 1000
    grid = N // BLK

    def body(x_ref, wt_ref, si_ref, ss_ref, out_ref):
        h = jnp.dot(x_ref[...], wt_ref[...], preferred_element_type=jnp.float32)
        out_ref[...] = jnp.maximum(
            si_ref[...] + ss_ref[...] + h * (1.0 + 1e-06), 0.0)

    full = pl.BlockSpec((D, D), lambda i: (0, 0))
    rows = pl.BlockSpec((BLK, D), lambda i: (i, 0))
    return pl.pallas_call(
        body,
        grid=(grid,),
        in_specs=[rows, full, rows, rows],
        out_specs=rows,
        out_shape=jax.ShapeDtypeStruct((N, D), jnp.float32),
    )(x, w_har_t, s_irr, s_sol)


def _sc_conv_pair(xm_irr, xm_sol, ais, ait, ass, ast,
                  edl, edu, chunks_per_tile):
    T = chunks_per_tile

    def process(xm_hbm, asrc_hbm, atgt_hbm, ed_hbm,
                out_hbm, acc, a_src_v, a_tgt_v,
                ed0, src0, sidx0, rows0,
                ed1, src1, sidx1, rows1,
                w_v, sem_g0, sem_g1, sem_s0, sem_s1,
                sem_i0, sem_i1, tile):
        pltpu.sync_copy(asrc_hbm, a_src_v)
        pltpu.sync_copy(atgt_hbm, a_tgt_v)

        z16 = jnp.zeros((LANES,), jnp.float32)

        def zrow(r, carry):
            for j in range(D // LANES):
                rows0[r, pl.ds(j * LANES, LANES)] = z16
            return carry

        lax.fori_loop(0, ROWBLK, zrow, 0)
        for i in range(8):
            b = tile + i * NUM_TILES
            @pl.when(b < NUM_ROWBLKS)
            def _():
                pltpu.sync_copy(rows0.at[pl.ds(0, ROWBLK)],
                                acc.at[pl.ds(b * ROWBLK, ROWBLK)])
        plsc.subcore_barrier()

        def start_idx(c, ed_v, sem):
            pltpu.async_copy(ed_hbm.at[tile * T + c], ed_v, sem)

        def wait_idx(c, ed_v, sem):
            pltpu.make_async_copy(ed_hbm.at[tile * T + c], ed_v, sem).wait()

        def extract_src(ed_v, s_v):
            for g in range(CHUNK // LANES):
                sl = pl.ds(g * LANES, LANES)
                s_v[sl] = ed_v[1, sl]

        def compute_w(ed_v, s_v, sidx):
            for g in range(CHUNK // LANES):
                sl = pl.ds(g * LANES, LANES)
                t = ed_v[0, sl]
                za = (plsc.load_gather(a_src_v, [s_v[sl]])
                      + plsc.load_gather(a_tgt_v, [t]))
                att = jnp.where(za > 0.0, za,
                                jnp.exp(jnp.minimum(za, 0.0)) - 1.0)
                w_v[sl] = att * plsc.bitcast(ed_v[2, sl], jnp.float32)
                sidx[sl] = t

        def scale_rows(rows):
            def scale(g, c2):
                wvec = w_v[pl.ds(g * LANES, LANES)]
                for k2 in range(LANES):
                    wb = jnp.full((LANES,), wvec[k2], jnp.float32)
                    k = g * LANES + k2
                    for j in range(D // LANES):
                        sj = pl.ds(j * LANES, LANES)
                        rows[k, sj] = rows[k, sj] * wb
                return c2
            lax.fori_loop(0, CHUNK // LANES, scale, 0)

        start_idx(0, ed0, sem_i0)
        start_idx(1, ed1, sem_i1)
        wait_idx(0, ed0, sem_i0)
        extract_src(ed0, src0)
        pltpu.async_copy(xm_hbm.at[src0], rows0, sem_g0)

        def pipe_body(i, carry):
            @pl.when(i > 0)
            def _():
                pltpu.make_async_copy(rows1, acc.at[sidx1], sem_s1).wait()
            wait_idx(2 * i + 1, ed1, sem_i1)
            extract_src(ed1, src1)
            pltpu.async_copy(xm_hbm.at[src1], rows1, sem_g1)
            compute_w(ed0, src0, sidx0)
            pltpu.make_async_copy(xm_hbm.at[src0], rows0, sem_g0).wait()
            scale_rows(rows0)
            pltpu.async_copy(rows0, acc.at[sidx0], sem_s0, add=True)

            @pl.when(i < T // 2 - 1)
            def _():
                start_idx(2 * i + 2, ed0, sem_i0)

            pltpu.make_async_copy(rows0, acc.at[sidx0], sem_s0).wait()

            @pl.when(i < T // 2 - 1)
            def _():
                wait_idx(2 * i + 2, ed0, sem_i0)
                extract_src(ed0, src0)
                pltpu.async_copy(xm_hbm.at[src0], rows0, sem_g0)
            compute_w(ed1, src1, sidx1)
            pltpu.make_async_copy(xm_hbm.at[src1], rows1, sem_g1).wait()
            scale_rows(rows1)
            pltpu.async_copy(rows1, acc.at[sidx1], sem_s1, add=True)

            @pl.when(i < T // 2 - 1)
            def _():
                start_idx(2 * i + 3, ed1, sem_i1)
            return carry

        lax.fori_loop(0, T // 2, pipe_body, 0)
        pltpu.make_async_copy(rows1, acc.at[sidx1], sem_s1).wait()
        plsc.subcore_barrier()

        for i in range(8):
            b = tile + i * NUM_TILES
            @pl.when(b < NUM_ROWBLKS)
            def _():
                pltpu.sync_copy(acc.at[pl.ds(b * ROWBLK, ROWBLK)],
                                rows0.at[pl.ds(0, ROWBLK)])
                pltpu.sync_copy(rows0.at[pl.ds(0, ROWBLK)],
                                out_hbm.at[pl.ds(b * ROWBLK, ROWBLK)])

    def body(xm_irr_h, xm_sol_h, ais_h, ait_h, ass_h, ast_h,
             edl_h, edu_h,
             out_irr, out_sol, acc, a_src_v, a_tgt_v,
             ed0, src0, sidx0, rows0,
             ed1, src1, sidx1, rows1,
             w_v, sem_g0, sem_g1, sem_s0, sem_s1, sem_i0, sem_i1):
        c = lax.axis_index("c")
        tile = lax.axis_index("s")

        @pl.when(c == 0)
        def _():
            process(xm_irr_h, ais_h, ait_h, edl_h, out_irr,
                    acc, a_src_v, a_tgt_v,
                    ed0, src0, sidx0, rows0,
                    ed1, src1, sidx1, rows1,
                    w_v, sem_g0, sem_g1, sem_s0, sem_s1,
                    sem_i0, sem_i1, tile)

        @pl.when(c == 1)
        def _():
            process(xm_sol_h, ass_h, ast_h, edu_h, out_sol,
                    acc, a_src_v, a_tgt_v,
                    ed0, src0, sidx0, rows0,
                    ed1, src1, sidx1, rows1,
                    w_v, sem_g0, sem_g1, sem_s0, sem_s1,
                    sem_i0, sem_i1, tile)

    mesh = plsc.VectorSubcoreMesh(core_axis_name="c", subcore_axis_name="s")
    f = pl.kernel(
        body,
        out_type=[
            jax.ShapeDtypeStruct((N, D), jnp.float32),
            jax.ShapeDtypeStruct((N, D), jnp.float32),
        ],
        mesh=mesh,
        compiler_params=pltpu.CompilerParams(needs_layout_passes=False),
        scratch_types=[
            pltpu.VMEM_SHARED((N, D), jnp.float32),
            pltpu.VMEM((N,), jnp.float32),
            pltpu.VMEM((N,), jnp.float32),
            pltpu.VMEM((3, CHUNK), jnp.int32),
            pltpu.VMEM((CHUNK,), jnp.int32),
            pltpu.VMEM((CHUNK,), jnp.int32),
            pltpu.VMEM((CHUNK, D), jnp.float32),
            pltpu.VMEM((3, CHUNK), jnp.int32),
            pltpu.VMEM((CHUNK,), jnp.int32),
            pltpu.VMEM((CHUNK,), jnp.int32),
            pltpu.VMEM((CHUNK, D), jnp.float32),
            pltpu.VMEM((CHUNK,), jnp.float32),
            pltpu.SemaphoreType.DMA,
            pltpu.SemaphoreType.DMA,
            pltpu.SemaphoreType.DMA,
            pltpu.SemaphoreType.DMA,
            pltpu.SemaphoreType.DMA,
            pltpu.SemaphoreType.DMA,
        ],
    )
    return f(xm_irr, xm_sol, ais, ait, ass, ast, edl, edu)


def kernel(x, lower_neighborhood_indices, lower_neighborhood_values,
           upper_neighborhood_indices, upper_neighborhood_values,
           W_irr, att_irr, W_sol, att_sol, W_har):
    E = lower_neighborhood_values.shape[0]
    chunks_per_tile = -(-E // (NUM_TILES * CHUNK))
    chunks_per_tile += chunks_per_tile % 2
    per_tile = chunks_per_tile * CHUNK
    EP = per_tile * NUM_TILES
    pad = EP - E

    xm_irr, xm_sol, ais, ait, ass, ast = _stage1(
        x, W_irr, att_irr, W_sol, att_sol)

    def prep(idx, vals):
        t = jnp.pad(idx[0], (0, pad))
        s = jnp.pad(idx[1], (0, pad))
        v = lax.bitcast_convert_type(jnp.pad(vals, (0, pad)), jnp.int32)
        ed = jnp.stack([t, s, v]).reshape(3, EP // CHUNK, CHUNK)
        return ed.transpose(1, 0, 2)

    edl = prep(lower_neighborhood_indices, lower_neighborhood_values)
    edu = prep(upper_neighborhood_indices, upper_neighborhood_values)

    s_irr, s_sol = _sc_conv_pair(
        xm_irr, xm_sol,
        ais.reshape(N), ait.reshape(N), ass.reshape(N), ast.reshape(N),
        edl, edu, per_tile // CHUNK)

    return _stage3(x, W_har.T, s_irr, s_sol)
